# Initial kernel scaffold; baseline (speedup 1.0000x reference)
import jax
import jax.numpy as jnp
from jax.experimental import pallas as pl


def kernel(slot_feat, meta_feat, uvcc, rank, uvcc_table, rank_table, gamma, beta, W, b):
    raise NotImplementedError("write your pallas kernel here")


# speedup vs baseline: 2.2922x; 2.2922x over previous
import functools

import jax
import jax.numpy as jnp
from jax import lax
from jax.experimental import pallas as pl
from jax.experimental.pallas import tpu as pltpu
from jax.experimental.pallas import tpu_sc as plsc

_NB = 16


def _sc_gather(table, idx):
    V, D = table.shape
    Bn = idx.shape[0]
    NC, NS = 2, 16
    NW = NC * NS
    bpw = Bn // NW
    mesh = plsc.VectorSubcoreMesh(core_axis_name="c", subcore_axis_name="s")

    @functools.partial(
        pl.kernel,
        mesh=mesh,
        out_type=jax.ShapeDtypeStruct((Bn, D), table.dtype),
        scratch_types=[
            pltpu.VMEM((bpw,), jnp.int32),
            pltpu.VMEM((bpw, D), jnp.float32),
            pltpu.SemaphoreType.DMA,
        ],
        compiler_params=pltpu.CompilerParams(use_tc_tiling_on_sc=False),
    )
    def gk(table_hbm, idx_hbm, out_hbm, idx_v, rows_v, sem):
        wid = lax.axis_index("s") * NC + lax.axis_index("c")
        base = wid * bpw
        pltpu.sync_copy(idx_hbm.at[pl.ds(base, bpw)], idx_v)
        pltpu.async_copy(table_hbm.at[idx_v], rows_v, sem).wait()
        pltpu.sync_copy(rows_v, out_hbm.at[pl.ds(base, bpw)])

    return gk(table, idx)


def _stats_body(nt, ds, slot_ref, meta_ref, gamma_ref, beta_ref, w_ref, b_ref,
                scale_out, mnou_out, acc_ref):
    i = pl.program_id(0)
    n = pl.num_programs(0)

    @pl.when(i == 0)
    def _init():
        acc_ref[...] = jnp.zeros_like(acc_ref)

    x = slot_ref[...]
    acc_ref[0:1, :] += jnp.sum(x, axis=0, keepdims=True)
    acc_ref[1:2, :] += jnp.sum(x * x, axis=0, keepdims=True)

    @pl.when(i == n - 1)
    def _finalize():
        meta = meta_ref[...]
        bsz = meta.shape[0]
        gam = gamma_ref[...]
        bet = beta_ref[...]
        w = w_ref[...]

        mean_s = acc_ref[0:1, :] / nt
        var_s = acc_ref[1:2, :] / nt - mean_s * mean_s
        mean_m = jnp.sum(meta, axis=0, keepdims=True) / bsz
        var_m = jnp.sum(meta * meta, axis=0, keepdims=True) / bsz - mean_m * mean_m

        scale_s = gam[:, :ds] * lax.rsqrt(var_s + 1e-3)
        scale_m = gam[:, ds:] * lax.rsqrt(var_m + 1e-3)
        shift_s = bet[:, :ds] - mean_s * scale_s
        shift_m = bet[:, ds:] - mean_m * scale_m

        bsum = (
            jnp.dot(shift_s, w[:ds, :], preferred_element_type=jnp.float32)
            + jnp.dot(shift_m, w[ds:, :], preferred_element_type=jnp.float32)
            + b_ref[...]
        )
        mm = (meta * scale_m).astype(jnp.bfloat16)
        wm = w[ds:, :].astype(jnp.bfloat16)
        mnou_out[...] = (
            jnp.dot(mm, wm, preferred_element_type=jnp.float32) + bsum
        )
        scale_out[...] = scale_s


def _main_body(nb, t, ds, slot_ref, rank_ref, mnou_ref, u_ref, scale_ref,
               w_ref, rt_ref, out_ref):
    r_rows = nb * t
    x = slot_ref[...]
    xs = (x * scale_ref[...]).astype(jnp.bfloat16)
    ws = w_ref[...][:ds, :].astype(jnp.bfloat16)
    h = jnp.dot(xs, ws, preferred_element_type=jnp.float32)

    idx = rank_ref[...]
    idb = jnp.broadcast_to(idx, (r_rows, rt_ref.shape[0]))
    iot = lax.broadcasted_iota(jnp.int32, (r_rows, rt_ref.shape[0]), 1)
    oh = (idb == iot).astype(jnp.bfloat16)
    r = jnp.dot(oh, rt_ref[...], preferred_element_type=jnp.float32)

    mu = mnou_ref[...] + u_ref[...]
    mu3 = jnp.broadcast_to(mu[:, None, :], (nb, t, mu.shape[1]))
    out_ref[...] = h + r + mu3.reshape(r_rows, mu.shape[1])


def kernel(slot_feat, meta_feat, uvcc, rank, uvcc_table, rank_table, gamma,
           beta, W, b):
    B, T, DS = slot_feat.shape
    DM = meta_feat.shape[1]
    D = W.shape[1]
    f32 = jnp.float32
    R = _NB * T

    slot2d = slot_feat.reshape(B * T, DS)
    rank2d = rank.astype(jnp.int32).reshape(B * T, 1)
    gamma2 = gamma.reshape(1, DS + DM).astype(f32)
    beta2 = beta.reshape(1, DS + DM).astype(f32)
    b2 = b.reshape(1, D).astype(f32)
    n_cls = 256
    rt_bf = (
        jnp.zeros((n_cls, D), jnp.bfloat16)
        .at[: rank_table.shape[0]]
        .set(rank_table.astype(jnp.bfloat16))
    )

    u = _sc_gather(uvcc_table.astype(f32), uvcc.astype(jnp.int32))

    scale_s, mnou = pl.pallas_call(
        functools.partial(_stats_body, float(B * T), DS),
        grid=(B // _NB,),
        in_specs=[
            pl.BlockSpec((R, DS), lambda i: (i, 0)),
            pl.BlockSpec((B, DM), lambda i: (0, 0)),
            pl.BlockSpec((1, DS + DM), lambda i: (0, 0)),
            pl.BlockSpec((1, DS + DM), lambda i: (0, 0)),
            pl.BlockSpec((DS + DM, D), lambda i: (0, 0)),
            pl.BlockSpec((1, D), lambda i: (0, 0)),
        ],
        out_specs=[
            pl.BlockSpec((1, DS), lambda i: (0, 0)),
            pl.BlockSpec((B, D), lambda i: (0, 0)),
        ],
        out_shape=[
            jax.ShapeDtypeStruct((1, DS), f32),
            jax.ShapeDtypeStruct((B, D), f32),
        ],
        scratch_shapes=[pltpu.VMEM((2, DS), f32)],
    )(slot2d, meta_feat, gamma2, beta2, W, b2)

    out2d = pl.pallas_call(
        functools.partial(_main_body, _NB, T, DS),
        grid=(B // _NB,),
        in_specs=[
            pl.BlockSpec((R, DS), lambda i: (i, 0)),
            pl.BlockSpec((R, 1), lambda i: (i, 0)),
            pl.BlockSpec((_NB, D), lambda i: (i, 0)),
            pl.BlockSpec((_NB, D), lambda i: (i, 0)),
            pl.BlockSpec((1, DS), lambda i: (0, 0)),
            pl.BlockSpec((DS + DM, D), lambda i: (0, 0)),
            pl.BlockSpec((n_cls, D), lambda i: (0, 0)),
        ],
        out_specs=pl.BlockSpec((R, D), lambda i: (i, 0)),
        out_shape=jax.ShapeDtypeStruct((B * T, D), f32),
    )(slot2d, rank2d, mnou, u, scale_s, W, rt_bf)

    return out2d.reshape(B, T, D)
